# baseline (speedup 1.0000x reference)
import functools

import jax, jax.numpy as jnp
import numpy as np
from jax.experimental import pallas as pl
from jax.experimental.pallas import tpu as pltpu
from jax.experimental.pallas import tpu_sc as plsc

PATCH_SIZES = (16, 32, 48)
VOCAB = 4096
DIM = 32
BETA = 0.25
QUANT_RESI = 0.5
GROUPS = 8
RES_MAP = (0, 1, 2)

M_BLK = 3072
V_BLK = 1024


_G_FOR_PS = {16: 36, 32: 9, 48: 4}


def _conv9(x, w, b, ps, gb):
    T = x.shape[1]
    lanes = jax.lax.broadcasted_iota(jnp.int32, (1, T), 1)
    ph = lanes // ps
    pw = lanes % ps
    parts = []
    for di in (-1, 0, 1):
        for dj in (-1, 0, 1):
            shift = di * ps + dj
            xs = x if shift == 0 else jnp.roll(x, -shift, axis=1)
            valid = (ph + di >= 0) & (ph + di < ps) & (pw + dj >= 0) & (pw + dj < ps)
            parts.append(jnp.where(valid, xs, 0.0).astype(jnp.bfloat16))
    wb = w.astype(jnp.bfloat16)
    outs = []
    for g in range(gb):
        x9 = jnp.concatenate([p[g * DIM:(g + 1) * DIM] for p in parts], axis=0)
        outs.append(jax.lax.dot_general(wb, x9, (((1,), (0,)), ((), ())),
                                        preferred_element_type=jnp.float32))
    y = outs[0] if gb == 1 else jnp.concatenate(outs, axis=0)
    return y + jnp.tile(b, (gb, 1)) if gb > 1 else y + b


def _gn(y, g, be, gb):
    T = y.shape[1]
    n = jnp.float32(4 * T)
    r = jax.lax.broadcasted_iota(jnp.int32, (GROUPS * gb, DIM * gb), 0)
    c = jax.lax.broadcasted_iota(jnp.int32, (GROUPS * gb, DIM * gb), 1)
    sel = (((c // DIM) == (r // GROUPS)) &
           ((c % DIM) // 4 == (r % GROUPS))).astype(jnp.float32)
    selT = jnp.transpose(sel)
    hp = jax.lax.Precision.HIGHEST
    gs = jax.lax.dot_general(sel, y, (((1,), (0,)), ((), ())), precision=hp)
    m = jnp.sum(gs, axis=1, keepdims=True) / n
    mc = jax.lax.dot_general(selT, m, (((1,), (0,)), ((), ())), precision=hp)
    cen = y - mc
    q = cen * cen
    qs = jax.lax.dot_general(sel, q, (((1,), (0,)), ((), ())), precision=hp)
    v = jnp.sum(qs, axis=1, keepdims=True) / n
    vc = jax.lax.dot_general(selT, v, (((1,), (0,)), ((), ())), precision=hp)
    xn = cen / jnp.sqrt(vc + 1e-5)
    gg = jnp.tile(g, (gb, 1)) if gb > 1 else g
    bb = jnp.tile(be, (gb, 1)) if gb > 1 else be
    return xn * gg + bb


def _conv_block_kern(ps, gb):
    def kern(x_ref, w1_ref, b1_ref, g_ref, be_ref, w2_ref, b2_ref, o_ref):
        x = x_ref[...]
        y = _conv9(x, w1_ref[...], b1_ref[...], ps, gb)
        y = _gn(y, g_ref[...], be_ref[...], gb)
        y = y * 0.5 * (1.0 + jax.lax.erf(y * np.float32(1.0 / np.sqrt(2.0))))
        y = _conv9(y, w2_ref[...], b2_ref[...], ps, gb)
        o_ref[...] = y
    return kern


def _conv_block(xcf, params, pref, ps):
    T = ps * ps
    P = xcf.shape[0] // DIM
    gb = _G_FOR_PS[ps]
    w1 = params[pref + '_w1'].transpose(0, 2, 3, 1).reshape(DIM, 9 * DIM)
    w2 = params[pref + '_w2'].transpose(0, 2, 3, 1).reshape(DIM, 9 * DIM)
    b1 = params[pref + '_b1'].reshape(DIM, 1)
    b2 = params[pref + '_b2'].reshape(DIM, 1)
    g = params[pref + '_g'].reshape(DIM, 1)
    be = params[pref + '_be'].reshape(DIM, 1)
    wspec = pl.BlockSpec((DIM, 9 * DIM), lambda i: (0, 0))
    vspec = pl.BlockSpec((DIM, 1), lambda i: (0, 0))
    return pl.pallas_call(
        _conv_block_kern(ps, gb),
        grid=(P // gb,),
        in_specs=[pl.BlockSpec((gb * DIM, T), lambda i: (i, 0)),
                  wspec, vspec, vspec, vspec, wspec, vspec],
        out_specs=pl.BlockSpec((gb * DIM, T), lambda i: (i, 0)),
        out_shape=jax.ShapeDtypeStruct((P * DIM, T), jnp.float32),
    )(xcf, w1, b1, g, be, w2, b2)


def _argmin_kernel(zf_ref, embT_ref, idx_ref):
    zb = zf_ref[...]
    zbb = zb.astype(jnp.bfloat16)
    c = jnp.sum(zb * zb, axis=1, keepdims=True)

    def body(t, carry):
        best_d, best_i = carry
        et = embT_ref[:, pl.ds(t * V_BLK, V_BLK)]
        e2 = jnp.sum(et * et, axis=0, keepdims=True)
        en = et.astype(jnp.bfloat16) * jnp.bfloat16(-2.0)
        mm = jax.lax.dot_general(zbb, en, (((1,), (0,)), ((), ())),
                                 preferred_element_type=jnp.float32)
        d = (c + e2) + mm
        tmin = jnp.min(d, axis=1, keepdims=True)
        lanes = jax.lax.broadcasted_iota(jnp.int32, d.shape, 1)
        tidx = jnp.min(jnp.where(d == tmin, lanes, VOCAB), axis=1, keepdims=True)
        upd = tmin < best_d
        best_i = jnp.where(upd, tidx + t * V_BLK, best_i)
        best_d = jnp.where(upd, tmin, best_d)
        return best_d, best_i

    init = (jnp.full((M_BLK, 1), jnp.inf, jnp.float32),
            jnp.zeros((M_BLK, 1), jnp.int32))
    _, best_i = jax.lax.fori_loop(0, VOCAB // V_BLK, body, init)
    idx_ref[...] = best_i


def _sc_gather(emb, idx):
    n = idx.shape[0]
    width = emb.shape[1]
    info = plsc.get_sparse_core_info()
    nw = info.num_cores * info.num_subcores
    b_per_w = n // nw
    mesh = plsc.VectorSubcoreMesh(core_axis_name="c", subcore_axis_name="s")

    @functools.partial(
        pl.kernel, mesh=mesh,
        out_type=jax.ShapeDtypeStruct((n, width), jnp.float32),
        scratch_types=[
            pltpu.VMEM((b_per_w,), jnp.int32),
            pltpu.VMEM((b_per_w, width), jnp.float32),
            pltpu.SemaphoreType.DMA,
        ],
    )
    def k(table_hbm, idx_hbm, out_hbm, idx_v, rows_v, sem):
        wid = jax.lax.axis_index("s") * info.num_cores + jax.lax.axis_index("c")
        base = wid * b_per_w
        pltpu.sync_copy(idx_hbm.at[pl.ds(base, b_per_w)], idx_v)
        pltpu.async_copy(table_hbm.at[idx_v], rows_v, sem).wait()
        pltpu.sync_copy(rows_v, out_hbm.at[pl.ds(base, b_per_w)])

    return k(emb, idx)


def _res_block_kern(ps, gb):
    def kern(zq_ref, z_ref, w1_ref, b1_ref, g_ref, be_ref, w2_ref, b2_ref,
             o_ref, loss_ref):
        zq = zq_ref[...]
        z = z_ref[...]
        psum = jnp.sum((zq - z) ** 2).reshape(1, 1)
        zq_ste = z + (zq - z)
        y = _conv9(zq_ste, w1_ref[...], b1_ref[...], ps, gb)
        y = _gn(y, g_ref[...], be_ref[...], gb)
        y = y * 0.5 * (1.0 + jax.lax.erf(y * np.float32(1.0 / np.sqrt(2.0))))
        y = _conv9(y, w2_ref[...], b2_ref[...], ps, gb)
        o_ref[...] = zq_ste * QUANT_RESI + y * QUANT_RESI

        @pl.when(pl.program_id(0) == 0)
        def _():
            loss_ref[...] = psum

        @pl.when(pl.program_id(0) != 0)
        def _():
            loss_ref[...] = loss_ref[...] + psum
    return kern


def _res_block(zq, zcf, params, pref, ps):
    T = ps * ps
    P = zq.shape[0] // DIM
    gb = _G_FOR_PS[ps]
    w1 = params[pref + '_w1'].transpose(0, 2, 3, 1).reshape(DIM, 9 * DIM)
    w2 = params[pref + '_w2'].transpose(0, 2, 3, 1).reshape(DIM, 9 * DIM)
    b1 = params[pref + '_b1'].reshape(DIM, 1)
    b2 = params[pref + '_b2'].reshape(DIM, 1)
    g = params[pref + '_g'].reshape(DIM, 1)
    be = params[pref + '_be'].reshape(DIM, 1)
    wspec = pl.BlockSpec((DIM, 9 * DIM), lambda i: (0, 0))
    vspec = pl.BlockSpec((DIM, 1), lambda i: (0, 0))
    bspec = pl.BlockSpec((gb * DIM, T), lambda i: (i, 0))
    out, lsum = pl.pallas_call(
        _res_block_kern(ps, gb),
        grid=(P // gb,),
        in_specs=[bspec, bspec, wspec, vspec, vspec, vspec, wspec, vspec],
        out_specs=[bspec, pl.BlockSpec((1, 1), lambda i: (0, 0))],
        out_shape=[jax.ShapeDtypeStruct((P * DIM, T), jnp.float32),
                   jax.ShapeDtypeStruct((1, 1), jnp.float32)],
    )(zq, zcf, w1, b1, g, be, w2, b2)
    return out, lsum[0, 0]


def _codebook_argmin(zf, embT):
    n = zf.shape[0]
    idx = pl.pallas_call(
        _argmin_kernel,
        grid=(n // M_BLK,),
        in_specs=[
            pl.BlockSpec((M_BLK, DIM), lambda i: (i, 0)),
            pl.BlockSpec((DIM, VOCAB), lambda i: (0, 0)),
        ],
        out_specs=pl.BlockSpec((M_BLK, 1), lambda i: (i, 0)),
        out_shape=jax.ShapeDtypeStruct((n, 1), jnp.int32),
    )(zf, embT)
    return idx[:, 0]


def kernel(x, params):
    B, C, H, W = x.shape
    accumulation = jnp.zeros_like(x)
    total_loss = jnp.float32(0.0)
    indices_list = []
    emb = params['embedding']
    embT = emb.T
    emb_pad = jnp.pad(emb, ((0, 0), (0, 128 - DIM)))
    for s, ps in enumerate(PATCH_SIZES):
        n_h, n_w = H // ps, W // ps
        N = n_h * n_w
        P = B * N
        T = ps * ps
        resid = x - accumulation
        patches_cf = resid.reshape(B, C, n_h, ps, n_w, ps).transpose(0, 2, 4, 1, 3, 5).reshape(P * C, T)
        zcf = _conv_block(patches_cf, params, 'pre%d' % s, ps)
        zf = zcf.reshape(-1, DIM)
        idxs = _codebook_argmin(zf, embT)
        z_q = _sc_gather(emb_pad, idxs)[:, :DIM].reshape(P * C, T)
        indices_list.append(idxs.reshape(B, -1))
        out, lsum = _res_block(z_q, zcf, params, 'res%d' % RES_MAP[s], ps)
        m = lsum / jnp.float32(zf.shape[0] * DIM)
        total_loss = total_loss + (m + BETA * m)
        decoded = out.reshape(B, n_h, n_w, C, ps, ps).transpose(0, 3, 1, 4, 2, 5).reshape(B, C, H, W)
        accumulation = accumulation + decoded
    return jax.nn.sigmoid(accumulation), tuple(indices_list), total_loss


# speedup vs baseline: 1.0358x; 1.0358x over previous
import functools

import jax, jax.numpy as jnp
import numpy as np
from jax.experimental import pallas as pl
from jax.experimental.pallas import tpu as pltpu
from jax.experimental.pallas import tpu_sc as plsc

PATCH_SIZES = (16, 32, 48)
VOCAB = 4096
DIM = 32
BETA = 0.25
QUANT_RESI = 0.5
GROUPS = 8
RES_MAP = (0, 1, 2)

M_BLK = 2048
V_BLK = 2048


_G_FOR_PS = {16: 24, 32: 6, 48: 2}


def _conv9(x, w, b, ps, gb):
    T = x.shape[1]
    lanes = jax.lax.broadcasted_iota(jnp.int32, (1, T), 1)
    ph = lanes // ps
    pw = lanes % ps
    parts = []
    for di in (-1, 0, 1):
        for dj in (-1, 0, 1):
            shift = di * ps + dj
            xs = x if shift == 0 else jnp.roll(x, -shift, axis=1)
            valid = (ph + di >= 0) & (ph + di < ps) & (pw + dj >= 0) & (pw + dj < ps)
            parts.append(jnp.where(valid, xs, 0.0).astype(jnp.bfloat16))
    wb = w.astype(jnp.bfloat16)
    outs = []
    for g in range(gb):
        x9 = jnp.concatenate([p[g * DIM:(g + 1) * DIM] for p in parts], axis=0)
        outs.append(jax.lax.dot_general(wb, x9, (((1,), (0,)), ((), ())),
                                        preferred_element_type=jnp.float32))
    y = outs[0] if gb == 1 else jnp.concatenate(outs, axis=0)
    return y + jnp.tile(b, (gb, 1)) if gb > 1 else y + b


def _gn(y, g, be, gb):
    T = y.shape[1]
    n = jnp.float32(4 * T)
    r = jax.lax.broadcasted_iota(jnp.int32, (GROUPS * gb, DIM * gb), 0)
    c = jax.lax.broadcasted_iota(jnp.int32, (GROUPS * gb, DIM * gb), 1)
    sel = (((c // DIM) == (r // GROUPS)) &
           ((c % DIM) // 4 == (r % GROUPS))).astype(jnp.float32)
    selT = jnp.transpose(sel)
    hp = jax.lax.Precision.HIGHEST
    gs = jax.lax.dot_general(sel, y, (((1,), (0,)), ((), ())), precision=hp)
    m = jnp.sum(gs, axis=1, keepdims=True) / n
    mc = jax.lax.dot_general(selT, m, (((1,), (0,)), ((), ())), precision=hp)
    cen = y - mc
    q = cen * cen
    qs = jax.lax.dot_general(sel, q, (((1,), (0,)), ((), ())), precision=hp)
    v = jnp.sum(qs, axis=1, keepdims=True) / n
    vc = jax.lax.dot_general(selT, v, (((1,), (0,)), ((), ())), precision=hp)
    xn = cen / jnp.sqrt(vc + 1e-5)
    gg = jnp.tile(g, (gb, 1)) if gb > 1 else g
    bb = jnp.tile(be, (gb, 1)) if gb > 1 else be
    return xn * gg + bb


def _conv_block_kern(ps, gb):
    def kern(x_ref, w1_ref, b1_ref, g_ref, be_ref, w2_ref, b2_ref, o_ref):
        x = x_ref[...]
        y = _conv9(x, w1_ref[...], b1_ref[...], ps, gb)
        y = _gn(y, g_ref[...], be_ref[...], gb)
        y = y * 0.5 * (1.0 + jax.lax.erf(y * np.float32(1.0 / np.sqrt(2.0))))
        y = _conv9(y, w2_ref[...], b2_ref[...], ps, gb)
        o_ref[...] = y
    return kern


def _conv_block(xcf, params, pref, ps):
    T = ps * ps
    P = xcf.shape[0] // DIM
    gb = _G_FOR_PS[ps]
    w1 = params[pref + '_w1'].transpose(0, 2, 3, 1).reshape(DIM, 9 * DIM)
    w2 = params[pref + '_w2'].transpose(0, 2, 3, 1).reshape(DIM, 9 * DIM)
    b1 = params[pref + '_b1'].reshape(DIM, 1)
    b2 = params[pref + '_b2'].reshape(DIM, 1)
    g = params[pref + '_g'].reshape(DIM, 1)
    be = params[pref + '_be'].reshape(DIM, 1)
    wspec = pl.BlockSpec((DIM, 9 * DIM), lambda i: (0, 0))
    vspec = pl.BlockSpec((DIM, 1), lambda i: (0, 0))
    return pl.pallas_call(
        _conv_block_kern(ps, gb),
        grid=(P // gb,),
        in_specs=[pl.BlockSpec((gb * DIM, T), lambda i: (i, 0)),
                  wspec, vspec, vspec, vspec, wspec, vspec],
        out_specs=pl.BlockSpec((gb * DIM, T), lambda i: (i, 0)),
        out_shape=jax.ShapeDtypeStruct((P * DIM, T), jnp.float32),
    )(xcf, w1, b1, g, be, w2, b2)


def _argmin_kernel(zf_ref, embT_ref, idx_ref):
    zb = zf_ref[...]
    zbb = zb.astype(jnp.bfloat16)
    c = jnp.sum(zb * zb, axis=1, keepdims=True)

    def body(t, carry):
        best_d, best_i = carry
        et = embT_ref[:, pl.ds(t * V_BLK, V_BLK)]
        e2 = jnp.sum(et * et, axis=0, keepdims=True)
        en = et.astype(jnp.bfloat16) * jnp.bfloat16(-2.0)
        mm = jax.lax.dot_general(zbb, en, (((1,), (0,)), ((), ())),
                                 preferred_element_type=jnp.float32)
        d = (c + e2) + mm
        tmin = jnp.min(d, axis=1, keepdims=True)
        lanes = jax.lax.broadcasted_iota(jnp.int32, d.shape, 1)
        tidx = jnp.min(jnp.where(d == tmin, lanes, VOCAB), axis=1, keepdims=True)
        upd = tmin < best_d
        best_i = jnp.where(upd, tidx + t * V_BLK, best_i)
        best_d = jnp.where(upd, tmin, best_d)
        return best_d, best_i

    init = (jnp.full((M_BLK, 1), jnp.inf, jnp.float32),
            jnp.zeros((M_BLK, 1), jnp.int32))
    _, best_i = jax.lax.fori_loop(0, VOCAB // V_BLK, body, init)
    idx_ref[...] = best_i


def _sc_gather(emb, idx):
    n = idx.shape[0]
    width = emb.shape[1]
    info = plsc.get_sparse_core_info()
    nw = info.num_cores * info.num_subcores
    b_per_w = n // nw
    mesh = plsc.VectorSubcoreMesh(core_axis_name="c", subcore_axis_name="s")

    @functools.partial(
        pl.kernel, mesh=mesh,
        out_type=jax.ShapeDtypeStruct((n, width), jnp.float32),
        scratch_types=[
            pltpu.VMEM((b_per_w,), jnp.int32),
            pltpu.VMEM((b_per_w, width), jnp.float32),
            pltpu.SemaphoreType.DMA,
        ],
    )
    def k(table_hbm, idx_hbm, out_hbm, idx_v, rows_v, sem):
        wid = jax.lax.axis_index("s") * info.num_cores + jax.lax.axis_index("c")
        base = wid * b_per_w
        pltpu.sync_copy(idx_hbm.at[pl.ds(base, b_per_w)], idx_v)
        pltpu.async_copy(table_hbm.at[idx_v], rows_v, sem).wait()
        pltpu.sync_copy(rows_v, out_hbm.at[pl.ds(base, b_per_w)])

    return k(emb, idx)


def _res_block_kern(ps, gb):
    def kern(zq_ref, z_ref, w1_ref, b1_ref, g_ref, be_ref, w2_ref, b2_ref,
             o_ref, loss_ref):
        zq = zq_ref[...]
        z = z_ref[...]
        psum = jnp.sum((zq - z) ** 2).reshape(1, 1)
        zq_ste = z + (zq - z)
        y = _conv9(zq_ste, w1_ref[...], b1_ref[...], ps, gb)
        y = _gn(y, g_ref[...], be_ref[...], gb)
        y = y * 0.5 * (1.0 + jax.lax.erf(y * np.float32(1.0 / np.sqrt(2.0))))
        y = _conv9(y, w2_ref[...], b2_ref[...], ps, gb)
        o_ref[...] = zq_ste * QUANT_RESI + y * QUANT_RESI

        @pl.when(pl.program_id(0) == 0)
        def _():
            loss_ref[...] = psum

        @pl.when(pl.program_id(0) != 0)
        def _():
            loss_ref[...] = loss_ref[...] + psum
    return kern


def _res_block(zq, zcf, params, pref, ps):
    T = ps * ps
    P = zq.shape[0] // DIM
    gb = _G_FOR_PS[ps]
    w1 = params[pref + '_w1'].transpose(0, 2, 3, 1).reshape(DIM, 9 * DIM)
    w2 = params[pref + '_w2'].transpose(0, 2, 3, 1).reshape(DIM, 9 * DIM)
    b1 = params[pref + '_b1'].reshape(DIM, 1)
    b2 = params[pref + '_b2'].reshape(DIM, 1)
    g = params[pref + '_g'].reshape(DIM, 1)
    be = params[pref + '_be'].reshape(DIM, 1)
    wspec = pl.BlockSpec((DIM, 9 * DIM), lambda i: (0, 0))
    vspec = pl.BlockSpec((DIM, 1), lambda i: (0, 0))
    bspec = pl.BlockSpec((gb * DIM, T), lambda i: (i, 0))
    out, lsum = pl.pallas_call(
        _res_block_kern(ps, gb),
        grid=(P // gb,),
        in_specs=[bspec, bspec, wspec, vspec, vspec, vspec, wspec, vspec],
        out_specs=[bspec, pl.BlockSpec((1, 1), lambda i: (0, 0))],
        out_shape=[jax.ShapeDtypeStruct((P * DIM, T), jnp.float32),
                   jax.ShapeDtypeStruct((1, 1), jnp.float32)],
    )(zq, zcf, w1, b1, g, be, w2, b2)
    return out, lsum[0, 0]


def _codebook_argmin(zf, embT):
    n = zf.shape[0]
    idx = pl.pallas_call(
        _argmin_kernel,
        grid=(n // M_BLK,),
        in_specs=[
            pl.BlockSpec((M_BLK, DIM), lambda i: (i, 0)),
            pl.BlockSpec((DIM, VOCAB), lambda i: (0, 0)),
        ],
        out_specs=pl.BlockSpec((M_BLK, 1), lambda i: (i, 0)),
        out_shape=jax.ShapeDtypeStruct((n, 1), jnp.int32),
    )(zf, embT)
    return idx[:, 0]


def kernel(x, params):
    B, C, H, W = x.shape
    accumulation = jnp.zeros_like(x)
    total_loss = jnp.float32(0.0)
    indices_list = []
    emb = params['embedding']
    embT = emb.T
    emb_pad = jnp.pad(emb, ((0, 0), (0, 128 - DIM)))
    for s, ps in enumerate(PATCH_SIZES):
        n_h, n_w = H // ps, W // ps
        N = n_h * n_w
        P = B * N
        T = ps * ps
        resid = x - accumulation
        patches_cf = resid.reshape(B, C, n_h, ps, n_w, ps).transpose(0, 2, 4, 1, 3, 5).reshape(P * C, T)
        zcf = _conv_block(patches_cf, params, 'pre%d' % s, ps)
        zf = zcf.reshape(-1, DIM)
        idxs = _codebook_argmin(zf, embT)
        z_q = _sc_gather(emb_pad, idxs)[:, :DIM].reshape(P * C, T)
        indices_list.append(idxs.reshape(B, -1))
        out, lsum = _res_block(z_q, zcf, params, 'res%d' % RES_MAP[s], ps)
        m = lsum / jnp.float32(zf.shape[0] * DIM)
        total_loss = total_loss + (m + BETA * m)
        decoded = out.reshape(B, n_h, n_w, C, ps, ps).transpose(0, 3, 1, 4, 2, 5).reshape(B, C, H, W)
        accumulation = accumulation + decoded
    return jax.nn.sigmoid(accumulation), tuple(indices_list), total_loss


# speedup vs baseline: 1.0696x; 1.0327x over previous
import functools

import jax, jax.numpy as jnp
import numpy as np
from jax.experimental import pallas as pl
from jax.experimental.pallas import tpu as pltpu
from jax.experimental.pallas import tpu_sc as plsc

PATCH_SIZES = (16, 32, 48)
VOCAB = 4096
DIM = 32
BETA = 0.25
QUANT_RESI = 0.5
GROUPS = 8
RES_MAP = (0, 1, 2)

M_BLK = 2048
V_BLK = 4096


_G_FOR_PS = {16: 24, 32: 6, 48: 2}


def _conv9(x, w, b, ps, gb):
    T = x.shape[1]
    lanes = jax.lax.broadcasted_iota(jnp.int32, (1, T), 1)
    ph = lanes // ps
    pw = lanes % ps
    parts = []
    for di in (-1, 0, 1):
        for dj in (-1, 0, 1):
            shift = di * ps + dj
            xs = x if shift == 0 else jnp.roll(x, -shift, axis=1)
            valid = (ph + di >= 0) & (ph + di < ps) & (pw + dj >= 0) & (pw + dj < ps)
            parts.append(jnp.where(valid, xs, 0.0).astype(jnp.bfloat16))
    wb = w.astype(jnp.bfloat16)
    outs = []
    for g in range(gb):
        x9 = jnp.concatenate([p[g * DIM:(g + 1) * DIM] for p in parts], axis=0)
        outs.append(jax.lax.dot_general(wb, x9, (((1,), (0,)), ((), ())),
                                        preferred_element_type=jnp.float32))
    y = outs[0] if gb == 1 else jnp.concatenate(outs, axis=0)
    return y + jnp.tile(b, (gb, 1)) if gb > 1 else y + b


def _gn(y, g, be, gb):
    T = y.shape[1]
    n = jnp.float32(4 * T)
    r = jax.lax.broadcasted_iota(jnp.int32, (GROUPS * gb, DIM * gb), 0)
    c = jax.lax.broadcasted_iota(jnp.int32, (GROUPS * gb, DIM * gb), 1)
    sel = (((c // DIM) == (r // GROUPS)) &
           ((c % DIM) // 4 == (r % GROUPS))).astype(jnp.float32)
    selT = jnp.transpose(sel)
    hp = jax.lax.Precision.HIGHEST
    gs = jax.lax.dot_general(sel, y, (((1,), (0,)), ((), ())), precision=hp)
    m = jnp.sum(gs, axis=1, keepdims=True) / n
    mc = jax.lax.dot_general(selT, m, (((1,), (0,)), ((), ())), precision=hp)
    cen = y - mc
    q = cen * cen
    qs = jax.lax.dot_general(sel, q, (((1,), (0,)), ((), ())), precision=hp)
    v = jnp.sum(qs, axis=1, keepdims=True) / n
    vc = jax.lax.dot_general(selT, v, (((1,), (0,)), ((), ())), precision=hp)
    xn = cen / jnp.sqrt(vc + 1e-5)
    gg = jnp.tile(g, (gb, 1)) if gb > 1 else g
    bb = jnp.tile(be, (gb, 1)) if gb > 1 else be
    return xn * gg + bb


def _conv_block_kern(ps, gb):
    def kern(x_ref, w1_ref, b1_ref, g_ref, be_ref, w2_ref, b2_ref, o_ref):
        x = x_ref[...]
        y = _conv9(x, w1_ref[...], b1_ref[...], ps, gb)
        y = _gn(y, g_ref[...], be_ref[...], gb)
        y = y * 0.5 * (1.0 + jax.lax.erf(y * np.float32(1.0 / np.sqrt(2.0))))
        y = _conv9(y, w2_ref[...], b2_ref[...], ps, gb)
        o_ref[...] = y
    return kern


def _conv_block(xcf, params, pref, ps):
    T = ps * ps
    P = xcf.shape[0] // DIM
    gb = _G_FOR_PS[ps]
    w1 = params[pref + '_w1'].transpose(0, 2, 3, 1).reshape(DIM, 9 * DIM)
    w2 = params[pref + '_w2'].transpose(0, 2, 3, 1).reshape(DIM, 9 * DIM)
    b1 = params[pref + '_b1'].reshape(DIM, 1)
    b2 = params[pref + '_b2'].reshape(DIM, 1)
    g = params[pref + '_g'].reshape(DIM, 1)
    be = params[pref + '_be'].reshape(DIM, 1)
    wspec = pl.BlockSpec((DIM, 9 * DIM), lambda i: (0, 0))
    vspec = pl.BlockSpec((DIM, 1), lambda i: (0, 0))
    return pl.pallas_call(
        _conv_block_kern(ps, gb),
        grid=(P // gb,),
        in_specs=[pl.BlockSpec((gb * DIM, T), lambda i: (i, 0)),
                  wspec, vspec, vspec, vspec, wspec, vspec],
        out_specs=pl.BlockSpec((gb * DIM, T), lambda i: (i, 0)),
        out_shape=jax.ShapeDtypeStruct((P * DIM, T), jnp.float32),
    )(xcf, w1, b1, g, be, w2, b2)


def _argmin_kernel(zf_ref, embT_ref, idx_ref):
    zb = zf_ref[...]
    zbb = zb.astype(jnp.bfloat16)
    c = jnp.sum(zb * zb, axis=1, keepdims=True)

    def body(t, carry):
        best_d, best_i = carry
        et = embT_ref[:, pl.ds(t * V_BLK, V_BLK)]
        e2 = jnp.sum(et * et, axis=0, keepdims=True)
        en = et.astype(jnp.bfloat16) * jnp.bfloat16(-2.0)
        mm = jax.lax.dot_general(zbb, en, (((1,), (0,)), ((), ())),
                                 preferred_element_type=jnp.float32)
        d = (c + e2) + mm
        tmin = jnp.min(d, axis=1, keepdims=True)
        lanes = jax.lax.broadcasted_iota(jnp.int32, d.shape, 1)
        tidx = jnp.min(jnp.where(d == tmin, lanes, VOCAB), axis=1, keepdims=True)
        upd = tmin < best_d
        best_i = jnp.where(upd, tidx + t * V_BLK, best_i)
        best_d = jnp.where(upd, tmin, best_d)
        return best_d, best_i

    init = (jnp.full((M_BLK, 1), jnp.inf, jnp.float32),
            jnp.zeros((M_BLK, 1), jnp.int32))
    _, best_i = jax.lax.fori_loop(0, VOCAB // V_BLK, body, init)
    idx_ref[...] = best_i


def _sc_gather(emb, idx):
    n = idx.shape[0]
    width = emb.shape[1]
    info = plsc.get_sparse_core_info()
    nw = info.num_cores * info.num_subcores
    b_per_w = n // nw
    mesh = plsc.VectorSubcoreMesh(core_axis_name="c", subcore_axis_name="s")

    @functools.partial(
        pl.kernel, mesh=mesh,
        out_type=jax.ShapeDtypeStruct((n, width), jnp.float32),
        scratch_types=[
            pltpu.VMEM((b_per_w,), jnp.int32),
            pltpu.VMEM((b_per_w, width), jnp.float32),
            pltpu.SemaphoreType.DMA,
        ],
    )
    def k(table_hbm, idx_hbm, out_hbm, idx_v, rows_v, sem):
        wid = jax.lax.axis_index("s") * info.num_cores + jax.lax.axis_index("c")
        base = wid * b_per_w
        pltpu.sync_copy(idx_hbm.at[pl.ds(base, b_per_w)], idx_v)
        pltpu.async_copy(table_hbm.at[idx_v], rows_v, sem).wait()
        pltpu.sync_copy(rows_v, out_hbm.at[pl.ds(base, b_per_w)])

    return k(emb, idx)


def _res_block_kern(ps, gb):
    def kern(zq_ref, z_ref, w1_ref, b1_ref, g_ref, be_ref, w2_ref, b2_ref,
             o_ref, loss_ref):
        zq = zq_ref[...]
        z = z_ref[...]
        psum = jnp.sum((zq - z) ** 2).reshape(1, 1)
        zq_ste = z + (zq - z)
        y = _conv9(zq_ste, w1_ref[...], b1_ref[...], ps, gb)
        y = _gn(y, g_ref[...], be_ref[...], gb)
        y = y * 0.5 * (1.0 + jax.lax.erf(y * np.float32(1.0 / np.sqrt(2.0))))
        y = _conv9(y, w2_ref[...], b2_ref[...], ps, gb)
        o_ref[...] = zq_ste * QUANT_RESI + y * QUANT_RESI

        @pl.when(pl.program_id(0) == 0)
        def _():
            loss_ref[...] = psum

        @pl.when(pl.program_id(0) != 0)
        def _():
            loss_ref[...] = loss_ref[...] + psum
    return kern


def _res_block(zq, zcf, params, pref, ps):
    T = ps * ps
    P = zq.shape[0] // DIM
    gb = _G_FOR_PS[ps]
    w1 = params[pref + '_w1'].transpose(0, 2, 3, 1).reshape(DIM, 9 * DIM)
    w2 = params[pref + '_w2'].transpose(0, 2, 3, 1).reshape(DIM, 9 * DIM)
    b1 = params[pref + '_b1'].reshape(DIM, 1)
    b2 = params[pref + '_b2'].reshape(DIM, 1)
    g = params[pref + '_g'].reshape(DIM, 1)
    be = params[pref + '_be'].reshape(DIM, 1)
    wspec = pl.BlockSpec((DIM, 9 * DIM), lambda i: (0, 0))
    vspec = pl.BlockSpec((DIM, 1), lambda i: (0, 0))
    bspec = pl.BlockSpec((gb * DIM, T), lambda i: (i, 0))
    out, lsum = pl.pallas_call(
        _res_block_kern(ps, gb),
        grid=(P // gb,),
        in_specs=[bspec, bspec, wspec, vspec, vspec, vspec, wspec, vspec],
        out_specs=[bspec, pl.BlockSpec((1, 1), lambda i: (0, 0))],
        out_shape=[jax.ShapeDtypeStruct((P * DIM, T), jnp.float32),
                   jax.ShapeDtypeStruct((1, 1), jnp.float32)],
    )(zq, zcf, w1, b1, g, be, w2, b2)
    return out, lsum[0, 0]


def _codebook_argmin(zf, embT):
    n = zf.shape[0]
    idx = pl.pallas_call(
        _argmin_kernel,
        grid=(n // M_BLK,),
        in_specs=[
            pl.BlockSpec((M_BLK, DIM), lambda i: (i, 0)),
            pl.BlockSpec((DIM, VOCAB), lambda i: (0, 0)),
        ],
        out_specs=pl.BlockSpec((M_BLK, 1), lambda i: (i, 0)),
        out_shape=jax.ShapeDtypeStruct((n, 1), jnp.int32),
    )(zf, embT)
    return idx[:, 0]


def kernel(x, params):
    B, C, H, W = x.shape
    accumulation = jnp.zeros_like(x)
    total_loss = jnp.float32(0.0)
    indices_list = []
    emb = params['embedding']
    embT = emb.T
    emb_pad = jnp.pad(emb, ((0, 0), (0, 128 - DIM)))
    for s, ps in enumerate(PATCH_SIZES):
        n_h, n_w = H // ps, W // ps
        N = n_h * n_w
        P = B * N
        T = ps * ps
        resid = x - accumulation
        patches_cf = resid.reshape(B, C, n_h, ps, n_w, ps).transpose(0, 2, 4, 1, 3, 5).reshape(P * C, T)
        zcf = _conv_block(patches_cf, params, 'pre%d' % s, ps)
        zf = zcf.reshape(-1, DIM)
        idxs = _codebook_argmin(zf, embT)
        z_q = _sc_gather(emb_pad, idxs)[:, :DIM].reshape(P * C, T)
        indices_list.append(idxs.reshape(B, -1))
        out, lsum = _res_block(z_q, zcf, params, 'res%d' % RES_MAP[s], ps)
        m = lsum / jnp.float32(zf.shape[0] * DIM)
        total_loss = total_loss + (m + BETA * m)
        decoded = out.reshape(B, n_h, n_w, C, ps, ps).transpose(0, 3, 1, 4, 2, 5).reshape(B, C, H, W)
        accumulation = accumulation + decoded
    return jax.nn.sigmoid(accumulation), tuple(indices_list), total_loss


# speedup vs baseline: 1.0714x; 1.0017x over previous
import functools

import jax, jax.numpy as jnp
import numpy as np
from jax.experimental import pallas as pl
from jax.experimental.pallas import tpu as pltpu
from jax.experimental.pallas import tpu_sc as plsc

PATCH_SIZES = (16, 32, 48)
VOCAB = 4096
DIM = 32
BETA = 0.25
QUANT_RESI = 0.5
GROUPS = 8
RES_MAP = (0, 1, 2)

M_BLK = 3072
V_BLK = 4096


_G_FOR_PS = {16: 24, 32: 6, 48: 2}


def _conv9(x, w, b, ps, gb):
    T = x.shape[1]
    lanes = jax.lax.broadcasted_iota(jnp.int32, (1, T), 1)
    ph = lanes // ps
    pw = lanes % ps
    parts = []
    for di in (-1, 0, 1):
        for dj in (-1, 0, 1):
            shift = di * ps + dj
            xs = x if shift == 0 else jnp.roll(x, -shift, axis=1)
            valid = (ph + di >= 0) & (ph + di < ps) & (pw + dj >= 0) & (pw + dj < ps)
            parts.append(jnp.where(valid, xs, 0.0).astype(jnp.bfloat16))
    wb = w.astype(jnp.bfloat16)
    outs = []
    for g in range(gb):
        x9 = jnp.concatenate([p[g * DIM:(g + 1) * DIM] for p in parts], axis=0)
        outs.append(jax.lax.dot_general(wb, x9, (((1,), (0,)), ((), ())),
                                        preferred_element_type=jnp.float32))
    y = outs[0] if gb == 1 else jnp.concatenate(outs, axis=0)
    return y + jnp.tile(b, (gb, 1)) if gb > 1 else y + b


def _gn(y, g, be, gb):
    T = y.shape[1]
    n = jnp.float32(4 * T)
    r = jax.lax.broadcasted_iota(jnp.int32, (GROUPS * gb, DIM * gb), 0)
    c = jax.lax.broadcasted_iota(jnp.int32, (GROUPS * gb, DIM * gb), 1)
    sel = (((c // DIM) == (r // GROUPS)) &
           ((c % DIM) // 4 == (r % GROUPS))).astype(jnp.float32)
    selT = jnp.transpose(sel)
    hp = jax.lax.Precision.HIGHEST
    gs = jax.lax.dot_general(sel, y, (((1,), (0,)), ((), ())), precision=hp)
    m = jnp.sum(gs, axis=1, keepdims=True) / n
    mc = jax.lax.dot_general(selT, m, (((1,), (0,)), ((), ())), precision=hp)
    cen = y - mc
    q = cen * cen
    qs = jax.lax.dot_general(sel, q, (((1,), (0,)), ((), ())), precision=hp)
    v = jnp.sum(qs, axis=1, keepdims=True) / n
    vc = jax.lax.dot_general(selT, v, (((1,), (0,)), ((), ())), precision=hp)
    xn = cen / jnp.sqrt(vc + 1e-5)
    gg = jnp.tile(g, (gb, 1)) if gb > 1 else g
    bb = jnp.tile(be, (gb, 1)) if gb > 1 else be
    return xn * gg + bb


def _conv_block_kern(ps, gb):
    def kern(x_ref, w1_ref, b1_ref, g_ref, be_ref, w2_ref, b2_ref, o_ref):
        x = x_ref[...]
        y = _conv9(x, w1_ref[...], b1_ref[...], ps, gb)
        y = _gn(y, g_ref[...], be_ref[...], gb)
        y = y * 0.5 * (1.0 + jax.lax.erf(y * np.float32(1.0 / np.sqrt(2.0))))
        y = _conv9(y, w2_ref[...], b2_ref[...], ps, gb)
        o_ref[...] = y
    return kern


def _conv_block(xcf, params, pref, ps):
    T = ps * ps
    P = xcf.shape[0] // DIM
    gb = _G_FOR_PS[ps]
    w1 = params[pref + '_w1'].transpose(0, 2, 3, 1).reshape(DIM, 9 * DIM)
    w2 = params[pref + '_w2'].transpose(0, 2, 3, 1).reshape(DIM, 9 * DIM)
    b1 = params[pref + '_b1'].reshape(DIM, 1)
    b2 = params[pref + '_b2'].reshape(DIM, 1)
    g = params[pref + '_g'].reshape(DIM, 1)
    be = params[pref + '_be'].reshape(DIM, 1)
    wspec = pl.BlockSpec((DIM, 9 * DIM), lambda i: (0, 0))
    vspec = pl.BlockSpec((DIM, 1), lambda i: (0, 0))
    return pl.pallas_call(
        _conv_block_kern(ps, gb),
        grid=(P // gb,),
        in_specs=[pl.BlockSpec((gb * DIM, T), lambda i: (i, 0)),
                  wspec, vspec, vspec, vspec, wspec, vspec],
        out_specs=pl.BlockSpec((gb * DIM, T), lambda i: (i, 0)),
        out_shape=jax.ShapeDtypeStruct((P * DIM, T), jnp.float32),
    )(xcf, w1, b1, g, be, w2, b2)


def _argmin_kernel(zf_ref, embT_ref, idx_ref):
    zb = zf_ref[...]
    zbb = zb.astype(jnp.bfloat16)
    c = jnp.sum(zb * zb, axis=1, keepdims=True)

    def body(t, carry):
        best_d, best_i = carry
        et = embT_ref[:, pl.ds(t * V_BLK, V_BLK)]
        e2 = jnp.sum(et * et, axis=0, keepdims=True)
        en = et.astype(jnp.bfloat16) * jnp.bfloat16(-2.0)
        mm = jax.lax.dot_general(zbb, en, (((1,), (0,)), ((), ())),
                                 preferred_element_type=jnp.float32)
        d = (c + e2) + mm
        tmin = jnp.min(d, axis=1, keepdims=True)
        lanes = jax.lax.broadcasted_iota(jnp.int32, d.shape, 1)
        tidx = jnp.min(jnp.where(d == tmin, lanes, VOCAB), axis=1, keepdims=True)
        upd = tmin < best_d
        best_i = jnp.where(upd, tidx + t * V_BLK, best_i)
        best_d = jnp.where(upd, tmin, best_d)
        return best_d, best_i

    init = (jnp.full((M_BLK, 1), jnp.inf, jnp.float32),
            jnp.zeros((M_BLK, 1), jnp.int32))
    _, best_i = jax.lax.fori_loop(0, VOCAB // V_BLK, body, init)
    idx_ref[...] = best_i


def _sc_gather(emb, idx):
    n = idx.shape[0]
    width = emb.shape[1]
    info = plsc.get_sparse_core_info()
    nw = info.num_cores * info.num_subcores
    b_per_w = n // nw
    mesh = plsc.VectorSubcoreMesh(core_axis_name="c", subcore_axis_name="s")

    @functools.partial(
        pl.kernel, mesh=mesh,
        out_type=jax.ShapeDtypeStruct((n, width), jnp.float32),
        scratch_types=[
            pltpu.VMEM((b_per_w,), jnp.int32),
            pltpu.VMEM((b_per_w, width), jnp.float32),
            pltpu.SemaphoreType.DMA,
        ],
    )
    def k(table_hbm, idx_hbm, out_hbm, idx_v, rows_v, sem):
        wid = jax.lax.axis_index("s") * info.num_cores + jax.lax.axis_index("c")
        base = wid * b_per_w
        pltpu.sync_copy(idx_hbm.at[pl.ds(base, b_per_w)], idx_v)
        pltpu.async_copy(table_hbm.at[idx_v], rows_v, sem).wait()
        pltpu.sync_copy(rows_v, out_hbm.at[pl.ds(base, b_per_w)])

    return k(emb, idx)


def _res_block_kern(ps, gb):
    def kern(zq_ref, z_ref, w1_ref, b1_ref, g_ref, be_ref, w2_ref, b2_ref,
             o_ref, loss_ref):
        zq = zq_ref[...]
        z = z_ref[...]
        psum = jnp.sum((zq - z) ** 2).reshape(1, 1)
        zq_ste = z + (zq - z)
        y = _conv9(zq_ste, w1_ref[...], b1_ref[...], ps, gb)
        y = _gn(y, g_ref[...], be_ref[...], gb)
        y = y * 0.5 * (1.0 + jax.lax.erf(y * np.float32(1.0 / np.sqrt(2.0))))
        y = _conv9(y, w2_ref[...], b2_ref[...], ps, gb)
        o_ref[...] = zq_ste * QUANT_RESI + y * QUANT_RESI

        @pl.when(pl.program_id(0) == 0)
        def _():
            loss_ref[...] = psum

        @pl.when(pl.program_id(0) != 0)
        def _():
            loss_ref[...] = loss_ref[...] + psum
    return kern


def _res_block(zq, zcf, params, pref, ps):
    T = ps * ps
    P = zq.shape[0] // DIM
    gb = _G_FOR_PS[ps]
    w1 = params[pref + '_w1'].transpose(0, 2, 3, 1).reshape(DIM, 9 * DIM)
    w2 = params[pref + '_w2'].transpose(0, 2, 3, 1).reshape(DIM, 9 * DIM)
    b1 = params[pref + '_b1'].reshape(DIM, 1)
    b2 = params[pref + '_b2'].reshape(DIM, 1)
    g = params[pref + '_g'].reshape(DIM, 1)
    be = params[pref + '_be'].reshape(DIM, 1)
    wspec = pl.BlockSpec((DIM, 9 * DIM), lambda i: (0, 0))
    vspec = pl.BlockSpec((DIM, 1), lambda i: (0, 0))
    bspec = pl.BlockSpec((gb * DIM, T), lambda i: (i, 0))
    out, lsum = pl.pallas_call(
        _res_block_kern(ps, gb),
        grid=(P // gb,),
        in_specs=[bspec, bspec, wspec, vspec, vspec, vspec, wspec, vspec],
        out_specs=[bspec, pl.BlockSpec((1, 1), lambda i: (0, 0))],
        out_shape=[jax.ShapeDtypeStruct((P * DIM, T), jnp.float32),
                   jax.ShapeDtypeStruct((1, 1), jnp.float32)],
    )(zq, zcf, w1, b1, g, be, w2, b2)
    return out, lsum[0, 0]


def _codebook_argmin(zf, embT):
    n = zf.shape[0]
    idx = pl.pallas_call(
        _argmin_kernel,
        grid=(n // M_BLK,),
        in_specs=[
            pl.BlockSpec((M_BLK, DIM), lambda i: (i, 0)),
            pl.BlockSpec((DIM, VOCAB), lambda i: (0, 0)),
        ],
        out_specs=pl.BlockSpec((M_BLK, 1), lambda i: (i, 0)),
        out_shape=jax.ShapeDtypeStruct((n, 1), jnp.int32),
    )(zf, embT)
    return idx[:, 0]


def kernel(x, params):
    B, C, H, W = x.shape
    accumulation = jnp.zeros_like(x)
    total_loss = jnp.float32(0.0)
    indices_list = []
    emb = params['embedding']
    embT = emb.T
    emb_pad = jnp.pad(emb, ((0, 0), (0, 128 - DIM)))
    for s, ps in enumerate(PATCH_SIZES):
        n_h, n_w = H // ps, W // ps
        N = n_h * n_w
        P = B * N
        T = ps * ps
        resid = x - accumulation
        patches_cf = resid.reshape(B, C, n_h, ps, n_w, ps).transpose(0, 2, 4, 1, 3, 5).reshape(P * C, T)
        zcf = _conv_block(patches_cf, params, 'pre%d' % s, ps)
        zf = zcf.reshape(-1, DIM)
        idxs = _codebook_argmin(zf, embT)
        z_q = _sc_gather(emb_pad, idxs)[:, :DIM].reshape(P * C, T)
        indices_list.append(idxs.reshape(B, -1))
        out, lsum = _res_block(z_q, zcf, params, 'res%d' % RES_MAP[s], ps)
        m = lsum / jnp.float32(zf.shape[0] * DIM)
        total_loss = total_loss + (m + BETA * m)
        decoded = out.reshape(B, n_h, n_w, C, ps, ps).transpose(0, 3, 1, 4, 2, 5).reshape(B, C, H, W)
        accumulation = accumulation + decoded
    return jax.nn.sigmoid(accumulation), tuple(indices_list), total_loss
